# Initial kernel scaffold; baseline (speedup 1.0000x reference)
import jax
import jax.numpy as jnp
from jax.experimental import pallas as pl


def kernel(x, edge_index, W1, b1, W2, b2, Wfc, bfc):
    raise NotImplementedError("write your pallas kernel here")


# speedup vs baseline: 4.4409x; 4.4409x over previous
import functools

import jax
import jax.numpy as jnp
from jax import lax
from jax.experimental import pallas as pl
from jax.experimental.pallas import tpu as pltpu
from jax.experimental.pallas import tpu_sc as plsc

N_NODES = 10000
N_EDGES = 320000
D_FEAT = 128

NP = 10240
CH = 128
NW = 32
NCH = 80
EP = NW * NCH * CH
ROWS_PER_SUB = NP // 16

_mesh = plsc.VectorSubcoreMesh(core_axis_name="c", subcore_axis_name="s")
_sc_params = pltpu.CompilerParams(use_tc_tiling_on_sc=False)


def _zero_vmem_2d(ref, rows, cols):
    zero = jnp.zeros((16,), jnp.float32)
    for i in range(rows):
        for j in range(cols // 16):
            ref[i, pl.ds(j * 16, 16)] = zero


@functools.partial(
    pl.kernel,
    mesh=_mesh,
    compiler_params=_sc_params,
    out_type=jax.ShapeDtypeStruct((2, NP), jnp.float32),
    scratch_types=[
        pltpu.VMEM((NCH, CH), jnp.int32),
        pltpu.VMEM((CH,), jnp.float32),
        pltpu.VMEM((ROWS_PER_SUB,), jnp.float32),
        pltpu.VMEM_SHARED((NP,), jnp.float32),
    ],
)
def _deg_kernel(dst2d_hbm, out_hbm, idx_v, ones_v, zb_v, acc_sh):
    cid = lax.axis_index("c")
    sid = lax.axis_index("s")
    wid = cid * 16 + sid

    pltpu.sync_copy(dst2d_hbm.at[pl.ds(wid * NCH, NCH)], idx_v)

    one = jnp.ones((16,), jnp.float32)
    zero = jnp.zeros((16,), jnp.float32)
    for j in range(CH // 16):
        ones_v[pl.ds(j * 16, 16)] = one
    for j in range(ROWS_PER_SUB // 16):
        zb_v[pl.ds(j * 16, 16)] = zero

    pltpu.sync_copy(zb_v, acc_sh.at[pl.ds(sid * ROWS_PER_SUB, ROWS_PER_SUB)])
    plsc.subcore_barrier()

    def body(j, carry):
        pltpu.sync_copy(ones_v, acc_sh.at[idx_v.at[j]], add=True)
        return carry

    lax.fori_loop(0, NCH, body, 0)
    plsc.subcore_barrier()

    pltpu.sync_copy(
        acc_sh.at[pl.ds(sid * ROWS_PER_SUB, ROWS_PER_SUB)],
        out_hbm.at[cid, pl.ds(sid * ROWS_PER_SUB, ROWS_PER_SUB)],
    )


def _make_edge_scatter(feat):
    @functools.partial(
        pl.kernel,
        mesh=_mesh,
        compiler_params=_sc_params,
        out_type=jax.ShapeDtypeStruct((2, NP, feat), jnp.float32),
        scratch_types=[
            pltpu.VMEM((NCH, CH), jnp.int32),
            pltpu.VMEM((NCH, CH), jnp.int32),
            pltpu.VMEM((CH, feat), jnp.float32),
            pltpu.VMEM((CH, feat), jnp.float32),
            pltpu.VMEM((64, feat), jnp.float32),
            pltpu.VMEM_SHARED((NP, feat), jnp.float32),
            pltpu.SemaphoreType.DMA,
            pltpu.SemaphoreType.DMA,
        ],
    )
    def edge_scatter(src2d_hbm, dst2d_hbm, g_hbm, out_hbm,
                     src_v, dst_v, rows0_v, rows1_v, zb_v, acc_sh, sem0, sem1):
        cid = lax.axis_index("c")
        sid = lax.axis_index("s")
        wid = cid * 16 + sid

        pltpu.sync_copy(src2d_hbm.at[pl.ds(wid * NCH, NCH)], src_v)
        pltpu.sync_copy(dst2d_hbm.at[pl.ds(wid * NCH, NCH)], dst_v)

        _zero_vmem_2d(zb_v, 64, feat)
        for k in range(ROWS_PER_SUB // 64):
            pltpu.sync_copy(zb_v, acc_sh.at[pl.ds(sid * ROWS_PER_SUB + k * 64, 64)])
        plsc.subcore_barrier()

        pltpu.async_copy(g_hbm.at[src_v.at[0]], rows0_v, sem0)

        def body(i, carry):
            j = i * 2
            pltpu.make_async_copy(g_hbm.at[src_v.at[j]], rows0_v, sem0).wait()
            pltpu.async_copy(g_hbm.at[src_v.at[j + 1]], rows1_v, sem1)
            pltpu.sync_copy(rows0_v, acc_sh.at[dst_v.at[j]], add=True)
            pltpu.make_async_copy(g_hbm.at[src_v.at[j + 1]], rows1_v, sem1).wait()

            @pl.when(j + 2 < NCH)
            def _():
                pltpu.async_copy(g_hbm.at[src_v.at[j + 2]], rows0_v, sem0)

            pltpu.sync_copy(rows1_v, acc_sh.at[dst_v.at[j + 1]], add=True)
            return carry

        lax.fori_loop(0, NCH // 2, body, 0)
        plsc.subcore_barrier()

        pltpu.sync_copy(
            acc_sh.at[pl.ds(sid * ROWS_PER_SUB, ROWS_PER_SUB)],
            out_hbm.at[cid, pl.ds(sid * ROWS_PER_SUB, ROWS_PER_SUB)],
        )

    return edge_scatter


_edge_scatter_32 = _make_edge_scatter(32)
_edge_scatter_64 = _make_edge_scatter(64)


def _tc_prep_body(degp_ref, x_ref, w1_ref, dinv_ref, g1_ref):
    deg = degp_ref[0, :] + degp_ref[1, :] + 1.0
    dinv = lax.rsqrt(deg)
    dinv_ref[...] = dinv[:, None]
    t1 = jnp.dot(x_ref[...], w1_ref[...], preferred_element_type=jnp.float32)
    g1_ref[...] = t1 * dinv[:, None]


def _tc_mid_body(sp_ref, g1_ref, dinv_ref, b1_ref, w2_ref, g2_ref):
    dinv = dinv_ref[...]
    h1 = jnp.maximum(dinv * (sp_ref[0] + sp_ref[1] + g1_ref[...]) + b1_ref[...], 0.0)
    g2_ref[...] = jnp.dot(h1, w2_ref[...], preferred_element_type=jnp.float32) * dinv


def _tc_fin_body(sp_ref, g2_ref, dinv_ref, b2_ref, wfc_ref, bfc_ref, pc1_ref, out_ref):
    dinv = dinv_ref[...]
    h2 = jnp.maximum(dinv * (sp_ref[0] + sp_ref[1] + g2_ref[...]) + b2_ref[...], 0.0)
    z = jnp.dot(h2, wfc_ref[...], preferred_element_type=jnp.float32)
    out_ref[...] = jax.nn.sigmoid(pc1_ref[...] * z + bfc_ref[...])


def _tc_call(body, out_shapes, *args):
    return pl.pallas_call(
        body,
        out_shape=[jax.ShapeDtypeStruct(s, jnp.float32) for s in out_shapes],
    )(*args)


def kernel(x, edge_index, W1, b1, W2, b2, Wfc, bfc):
    N = x.shape[0]

    pad_e = EP - N_EDGES
    fill = jnp.full((pad_e,), N, jnp.int32)
    src2d = jnp.concatenate([edge_index[0], fill]).reshape(EP // CH, CH)
    dst2d = jnp.concatenate([edge_index[1], fill]).reshape(EP // CH, CH)
    xp = jnp.pad(x, ((0, NP - N), (0, 0)))

    Xsub = x[:, :-2]
    Xc = Xsub - jnp.mean(Xsub, axis=0, keepdims=True)
    _, _, Vt = jnp.linalg.svd(Xc, full_matrices=False)
    pc1 = (Xc @ Vt[0])[:, None]
    pc1 = lax.stop_gradient(pc1)
    pc1p = jnp.pad(pc1, ((0, NP - N), (0, 0)))

    deg_parts = _deg_kernel(dst2d)
    dinv, g1 = _tc_call(_tc_prep_body, [(NP, 1), (NP, 32)],
                        deg_parts, xp, W1)

    s1 = _edge_scatter_32(src2d, dst2d, g1)
    (g2,) = _tc_call(_tc_mid_body, [(NP, 64)],
                     s1, g1, dinv, b1.reshape(1, 32), W2)

    s2 = _edge_scatter_64(src2d, dst2d, g2)
    (outp,) = _tc_call(_tc_fin_body, [(NP, 1)],
                       s2, g2, dinv, b2.reshape(1, 64), Wfc,
                       bfc.reshape(1, 1), pc1p)

    return outp[:N]
